# baseline (device time: 9225 ns/iter reference)
import jax
import jax.numpy as jnp
from jax import lax
from jax.experimental import pallas as pl
from jax.experimental.pallas import tpu as pltpu

DIAG = "xy"


def kernel(x):
    m_per, n = x.shape
    h = m_per // 2

    def body(x_ref, out_ref, send_sem1, recv_sem1, send_sem2, recv_sem2):
        my_x = lax.axis_index("x")
        my_y = lax.axis_index("y")
        my_z = lax.axis_index("z")
        peer_y = (my_x, 1 - my_y, my_z)
        if DIAG == "xy":
            peer_d = (1 - my_x, 1 - my_y, my_z)
        else:
            peer_d = (my_x, 1 - my_y, my_z ^ 1)

        barrier_sem = pltpu.get_barrier_semaphore()
        for peer in (peer_y, peer_d):
            pl.semaphore_signal(
                barrier_sem, inc=1, device_id=peer,
                device_id_type=pl.DeviceIdType.MESH,
            )
        pl.semaphore_wait(barrier_sem, 2)

        base = my_y * m_per
        rdma1 = pltpu.make_async_remote_copy(
            src_ref=x_ref.at[pl.ds(0, h)],
            dst_ref=out_ref.at[pl.ds(base, h)],
            send_sem=send_sem1,
            recv_sem=recv_sem1,
            device_id=peer_y,
            device_id_type=pl.DeviceIdType.MESH,
        )
        rdma2 = pltpu.make_async_remote_copy(
            src_ref=x_ref.at[pl.ds(h, h)],
            dst_ref=out_ref.at[pl.ds(base + h, h)],
            send_sem=send_sem2,
            recv_sem=recv_sem2,
            device_id=peer_d,
            device_id_type=pl.DeviceIdType.MESH,
        )
        rdma1.start()
        rdma2.start()

        out_ref[pl.ds(base, m_per), :] = x_ref[...]

        rdma1.wait()
        rdma2.wait()

    return pl.pallas_call(
        body,
        out_shape=jax.ShapeDtypeStruct((2 * m_per, n), x.dtype),
        in_specs=[pl.BlockSpec(memory_space=pltpu.VMEM)],
        out_specs=pl.BlockSpec(memory_space=pltpu.VMEM),
        scratch_shapes=[
            pltpu.SemaphoreType.DMA,
            pltpu.SemaphoreType.DMA,
            pltpu.SemaphoreType.DMA,
            pltpu.SemaphoreType.DMA,
        ],
        compiler_params=pltpu.CompilerParams(collective_id=0),
    )(x)


# device time: 9110 ns/iter; 1.0126x vs baseline; 1.0126x over previous
import jax
import jax.numpy as jnp
from jax import lax
from jax.experimental import pallas as pl
from jax.experimental.pallas import tpu as pltpu

DIAG = "yz"


def kernel(x):
    m_per, n = x.shape
    h = m_per // 2

    def body(x_ref, out_ref, send_sem1, recv_sem1, send_sem2, recv_sem2):
        my_x = lax.axis_index("x")
        my_y = lax.axis_index("y")
        my_z = lax.axis_index("z")
        peer_y = (my_x, 1 - my_y, my_z)
        if DIAG == "xy":
            peer_d = (1 - my_x, 1 - my_y, my_z)
        else:
            peer_d = (my_x, 1 - my_y, my_z ^ 1)

        barrier_sem = pltpu.get_barrier_semaphore()
        for peer in (peer_y, peer_d):
            pl.semaphore_signal(
                barrier_sem, inc=1, device_id=peer,
                device_id_type=pl.DeviceIdType.MESH,
            )
        pl.semaphore_wait(barrier_sem, 2)

        base = my_y * m_per
        rdma1 = pltpu.make_async_remote_copy(
            src_ref=x_ref.at[pl.ds(0, h)],
            dst_ref=out_ref.at[pl.ds(base, h)],
            send_sem=send_sem1,
            recv_sem=recv_sem1,
            device_id=peer_y,
            device_id_type=pl.DeviceIdType.MESH,
        )
        rdma2 = pltpu.make_async_remote_copy(
            src_ref=x_ref.at[pl.ds(h, h)],
            dst_ref=out_ref.at[pl.ds(base + h, h)],
            send_sem=send_sem2,
            recv_sem=recv_sem2,
            device_id=peer_d,
            device_id_type=pl.DeviceIdType.MESH,
        )
        rdma1.start()
        rdma2.start()

        out_ref[pl.ds(base, m_per), :] = x_ref[...]

        rdma1.wait()
        rdma2.wait()

    return pl.pallas_call(
        body,
        out_shape=jax.ShapeDtypeStruct((2 * m_per, n), x.dtype),
        in_specs=[pl.BlockSpec(memory_space=pltpu.VMEM)],
        out_specs=pl.BlockSpec(memory_space=pltpu.VMEM),
        scratch_shapes=[
            pltpu.SemaphoreType.DMA,
            pltpu.SemaphoreType.DMA,
            pltpu.SemaphoreType.DMA,
            pltpu.SemaphoreType.DMA,
        ],
        compiler_params=pltpu.CompilerParams(collective_id=0),
    )(x)


# device time: 4294 ns/iter; 2.1483x vs baseline; 2.1216x over previous
import os

import jax
import jax.numpy as jnp
from jax import lax
from jax.experimental import pallas as pl
from jax.experimental.pallas import tpu as pltpu

MODE = os.environ.get("KERNEL_MODE", "direct")


def kernel(x):
    m_per, n = x.shape

    def body(x_ref, out_ref, send_sem, recv_sem):
        my_x = lax.axis_index("x")
        my_y = lax.axis_index("y")
        my_z = lax.axis_index("z")
        peer = (my_x, 1 - my_y, my_z)

        if MODE != "nobar":
            barrier_sem = pltpu.get_barrier_semaphore()
            pl.semaphore_signal(
                barrier_sem, inc=1, device_id=peer,
                device_id_type=pl.DeviceIdType.MESH,
            )
            pl.semaphore_wait(barrier_sem, 1)

        base = my_y * m_per
        if MODE == "probe":
            out_ref[pl.ds(base, m_per), :] = x_ref[...]
            out_ref[pl.ds((1 - my_y) * m_per, m_per), :] = x_ref[...]
            return

        rdma = pltpu.make_async_remote_copy(
            src_ref=x_ref,
            dst_ref=out_ref.at[pl.ds(base, m_per)],
            send_sem=send_sem,
            recv_sem=recv_sem,
            device_id=peer,
            device_id_type=pl.DeviceIdType.MESH,
        )
        rdma.start()
        out_ref[pl.ds(base, m_per), :] = x_ref[...]
        rdma.wait()

    return pl.pallas_call(
        body,
        out_shape=jax.ShapeDtypeStruct((2 * m_per, n), x.dtype),
        in_specs=[pl.BlockSpec(memory_space=pltpu.VMEM)],
        out_specs=pl.BlockSpec(memory_space=pltpu.VMEM),
        scratch_shapes=[
            pltpu.SemaphoreType.DMA,
            pltpu.SemaphoreType.DMA,
        ],
        compiler_params=pltpu.CompilerParams(collective_id=0)
        if MODE != "nobar"
        else pltpu.CompilerParams(),
    )(x)
